# baseline (speedup 1.0000x reference)
import functools

import jax
import jax.numpy as jnp
from jax import lax
from jax.experimental import pallas as pl
from jax.experimental.pallas import tpu as pltpu
from jax.experimental.pallas import tpu_sc as plsc

_NUM_CORES = 2
_NUM_SUBCORES = 16
_NUM_WORKERS = _NUM_CORES * _NUM_SUBCORES


def _remap_body(b_per_w, o_per_w, keys_hbm, table_hbm, vals_hbm, offs_hbm,
                out_hbm, vals_out, offs_out, idx_v, rows_v, tab_sh, sem, sem2):
    s = lax.axis_index("s")
    wid = s * _NUM_CORES + lax.axis_index("c")
    base = wid * b_per_w
    obase = wid * o_per_w
    vcopy = pltpu.async_copy(
        vals_hbm.at[pl.ds(base, b_per_w)], vals_out.at[pl.ds(base, b_per_w)], sem2)
    ocopy = pltpu.async_copy(
        offs_hbm.at[pl.ds(obase, o_per_w)], offs_out.at[pl.ds(obase, o_per_w)], sem2)
    pltpu.sync_copy(keys_hbm.at[pl.ds(base, b_per_w)], idx_v)

    @pl.when(s == 0)
    def _stage_table():
        pltpu.sync_copy(table_hbm, tab_sh)

    plsc.subcore_barrier()
    pltpu.async_copy(tab_sh.at[idx_v], rows_v, sem).wait()
    pltpu.sync_copy(rows_v, out_hbm.at[pl.ds(base, b_per_w)])
    vcopy.wait()
    ocopy.wait()


def kernel(offsets, keys, values, id2index):
    total = keys.shape[0]
    batch = offsets.shape[0]
    b_per_w = total // _NUM_WORKERS
    o_per_w = batch // _NUM_WORKERS
    mesh = plsc.VectorSubcoreMesh(core_axis_name="c", subcore_axis_name="s")
    remap = pl.kernel(
        functools.partial(_remap_body, b_per_w, o_per_w),
        mesh=mesh,
        out_type=(
            jax.ShapeDtypeStruct((total,), jnp.int32),
            jax.ShapeDtypeStruct((total,), jnp.float32),
            jax.ShapeDtypeStruct((batch,), jnp.int32),
        ),
        scratch_types=[
            pltpu.VMEM((b_per_w,), jnp.int32),
            pltpu.VMEM((b_per_w,), jnp.int32),
            pltpu.VMEM_SHARED((id2index.shape[0],), jnp.int32),
            pltpu.SemaphoreType.DMA,
            pltpu.SemaphoreType.DMA,
        ],
    )
    idx_keys, vals_out, offs_out = remap(keys, id2index, values, offsets)
    return (offs_out, idx_keys, vals_out)


# speedup vs baseline: 1.7290x; 1.7290x over previous
import functools

import jax
import jax.numpy as jnp
from jax import lax
from jax.experimental import pallas as pl
from jax.experimental.pallas import tpu as pltpu
from jax.experimental.pallas import tpu_sc as plsc

_NUM_CORES = 2
_NUM_SUBCORES = 16
_NUM_WORKERS = _NUM_CORES * _NUM_SUBCORES


def _remap_body(b_per_w, keys_hbm, table_hbm, out_hbm, idx_v, rows_v, tab_sh, sem):
    s = lax.axis_index("s")
    wid = s * _NUM_CORES + lax.axis_index("c")
    base = wid * b_per_w
    pltpu.sync_copy(keys_hbm.at[pl.ds(base, b_per_w)], idx_v)

    @pl.when(s == 0)
    def _stage_table():
        pltpu.sync_copy(table_hbm, tab_sh)

    plsc.subcore_barrier()
    pltpu.async_copy(tab_sh.at[idx_v], rows_v, sem).wait()
    pltpu.sync_copy(rows_v, out_hbm.at[pl.ds(base, b_per_w)])


def kernel(offsets, keys, values, id2index):
    total = keys.shape[0]
    b_per_w = total // _NUM_WORKERS
    mesh = plsc.VectorSubcoreMesh(core_axis_name="c", subcore_axis_name="s")
    remap = pl.kernel(
        functools.partial(_remap_body, b_per_w),
        mesh=mesh,
        out_type=jax.ShapeDtypeStruct((total,), jnp.int32),
        scratch_types=[
            pltpu.VMEM((b_per_w,), jnp.int32),
            pltpu.VMEM((b_per_w,), jnp.int32),
            pltpu.VMEM_SHARED((id2index.shape[0],), jnp.int32),
            pltpu.SemaphoreType.DMA,
        ],
    )
    idx_keys = remap(keys, id2index)
    return (offsets, idx_keys, values)


# speedup vs baseline: 1.7385x; 1.0055x over previous
import functools

import jax
import jax.numpy as jnp
from jax import lax
from jax.experimental import pallas as pl
from jax.experimental.pallas import tpu as pltpu
from jax.experimental.pallas import tpu_sc as plsc

_NUM_CORES = 2
_NUM_SUBCORES = 16
_NUM_WORKERS = _NUM_CORES * _NUM_SUBCORES


_HBM_SPLIT = 1792


def _remap_body(b_per_w, vocab, keys_hbm, table_hbm, out_hbm, idx_v, rows_v,
                tab_sh, sem_h, sem_s):
    s = lax.axis_index("s")
    wid = s * _NUM_CORES + lax.axis_index("c")
    base = wid * b_per_w
    n_s = b_per_w - _HBM_SPLIT
    pltpu.sync_copy(keys_hbm.at[pl.ds(base, b_per_w)], idx_v)
    hcopy = pltpu.async_copy(
        table_hbm.at[idx_v.at[pl.ds(0, _HBM_SPLIT)]],
        rows_v.at[pl.ds(0, _HBM_SPLIT)], sem_h)
    del vocab

    @pl.when(s == 0)
    def _stage_table():
        pltpu.sync_copy(table_hbm, tab_sh)

    plsc.subcore_barrier()
    scopy = pltpu.async_copy(
        tab_sh.at[idx_v.at[pl.ds(_HBM_SPLIT, n_s)]],
        rows_v.at[pl.ds(_HBM_SPLIT, n_s)], sem_s)
    hcopy.wait()
    scopy.wait()
    pltpu.sync_copy(rows_v, out_hbm.at[pl.ds(base, b_per_w)])


def kernel(offsets, keys, values, id2index):
    total = keys.shape[0]
    b_per_w = total // _NUM_WORKERS
    mesh = plsc.VectorSubcoreMesh(core_axis_name="c", subcore_axis_name="s")
    remap = pl.kernel(
        functools.partial(_remap_body, b_per_w, id2index.shape[0]),
        mesh=mesh,
        out_type=jax.ShapeDtypeStruct((total,), jnp.int32),
        scratch_types=[
            pltpu.VMEM((b_per_w,), jnp.int32),
            pltpu.VMEM((b_per_w,), jnp.int32),
            pltpu.VMEM_SHARED((id2index.shape[0],), jnp.int32),
            pltpu.SemaphoreType.DMA,
            pltpu.SemaphoreType.DMA,
        ],
    )
    idx_keys = remap(keys, id2index)
    return (offsets, idx_keys, values)
